# Initial kernel scaffold; baseline (speedup 1.0000x reference)
import jax
import jax.numpy as jnp
from jax.experimental import pallas as pl


def kernel(det_boxes, det_scores, det_labels, boxes, labels):
    raise NotImplementedError("write your pallas kernel here")


# speedup vs baseline: 2.9525x; 2.9525x over previous
import jax
import jax.numpy as jnp
from jax.experimental import pallas as pl
from jax.experimental.pallas import tpu as pltpu

_OBJ_PAD = 1024
_DET_BLK = 800


def _loss_kernel(n_obj_ref, det_c_ref, det_l_ref, gt_c_ref, gt_l_ref,
                 out_ref, m_acc):
    i = pl.program_id(0)
    nblk = pl.num_programs(0)

    @pl.when(i == 0)
    def _init():
        m_acc[...] = jnp.full((1, _OBJ_PAD), -1.0, jnp.float32)

    dc = det_c_ref[...]
    dxmin = dc[:, 0:1]
    dymin = dc[:, 1:2]
    dxmax = dc[:, 2:3]
    dymax = dc[:, 3:4]
    gc = gt_c_ref[...]
    gxmin = gc[0:1, :]
    gymin = gc[1:2, :]
    gxmax = gc[2:3, :]
    gymax = gc[3:4, :]

    lox = jnp.maximum(dxmin, gxmin)
    loy = jnp.maximum(dymin, gymin)
    hix = jnp.minimum(dxmax, gxmax)
    hiy = jnp.minimum(dymax, gymax)
    iw = jnp.maximum(hix - lox, 0.0)
    ih = jnp.maximum(hiy - loy, 0.0)
    inter = iw * ih
    area_d = (dxmax - dxmin) * (dymax - dymin)
    area_g = (gxmax - gxmin) * (gymax - gymin)
    union = (area_d + area_g) - inter
    iou = inter / union
    match = det_l_ref[...] == gt_l_ref[...]
    masked = jnp.where(match, iou, -1.0)
    part = jnp.max(masked, axis=0, keepdims=True)
    m_acc[...] = jnp.maximum(m_acc[...], part)

    @pl.when(i == nblk - 1)
    def _fin():
        m = m_acc[...]
        obj_idx = jax.lax.broadcasted_iota(jnp.int32, (1, _OBJ_PAD), 1)
        valid = (m >= 0.0) & (obj_idx < n_obj_ref[0])
        w = valid.astype(jnp.float32)
        n = jnp.sum(w, keepdims=True)
        s = jnp.sum(w * (1.0 - jnp.maximum(m, 0.0)), keepdims=True)
        out_ref[...] = (s / n).reshape(1, 1)


def kernel(det_boxes, det_scores, det_labels, boxes, labels):
    del det_scores
    db = det_boxes[0].astype(jnp.float32)
    dl = det_labels[0].astype(jnp.int32).reshape(-1, 1)
    gb = boxes[0].astype(jnp.float32)
    gl = labels[0].astype(jnp.int32)
    n_det = db.shape[0]
    n_obj = gb.shape[0]

    gt_c = jnp.zeros((4, _OBJ_PAD), jnp.float32).at[:, :n_obj].set(gb.T)
    gt_l = jnp.full((1, _OBJ_PAD), -1, jnp.int32).at[0, :n_obj].set(gl)
    n_obj_arr = jnp.full((1,), n_obj, jnp.int32)

    nblk = pl.cdiv(n_det, _DET_BLK)
    out = pl.pallas_call(
        _loss_kernel,
        grid=(nblk,),
        in_specs=[
            pl.BlockSpec(memory_space=pltpu.SMEM),
            pl.BlockSpec((_DET_BLK, 4), lambda i: (i, 0)),
            pl.BlockSpec((_DET_BLK, 1), lambda i: (i, 0)),
            pl.BlockSpec((4, _OBJ_PAD), lambda i: (0, 0)),
            pl.BlockSpec((1, _OBJ_PAD), lambda i: (0, 0)),
        ],
        out_specs=pl.BlockSpec((1, 1), lambda i: (0, 0)),
        out_shape=jax.ShapeDtypeStruct((1, 1), jnp.float32),
        scratch_shapes=[pltpu.VMEM((1, _OBJ_PAD), jnp.float32)],
    )(n_obj_arr, db, dl, gt_c, gt_l)
    return out[0, 0]


# speedup vs baseline: 3.1547x; 1.0685x over previous
import jax
import jax.numpy as jnp
from jax.experimental import pallas as pl
from jax.experimental.pallas import tpu as pltpu

_OBJ_PAD = 1024
_DET_BLK = 2000


def _loss_kernel(n_obj_ref, det_c_ref, det_l_ref, gt_c_ref, gt_l_ref,
                 out_ref, m_acc):
    i = pl.program_id(0)
    nblk = pl.num_programs(0)

    @pl.when(i == 0)
    def _init():
        m_acc[...] = jnp.full((1, _OBJ_PAD), -1.0, jnp.float32)

    dc = det_c_ref[...]
    dxmin = dc[:, 0:1]
    dymin = dc[:, 1:2]
    dxmax = dc[:, 2:3]
    dymax = dc[:, 3:4]
    gc = gt_c_ref[...]
    gxmin = gc[0:1, :]
    gymin = gc[1:2, :]
    gxmax = gc[2:3, :]
    gymax = gc[3:4, :]

    lox = jnp.maximum(dxmin, gxmin)
    loy = jnp.maximum(dymin, gymin)
    hix = jnp.minimum(dxmax, gxmax)
    hiy = jnp.minimum(dymax, gymax)
    iw = jnp.maximum(hix - lox, 0.0)
    ih = jnp.maximum(hiy - loy, 0.0)
    inter = iw * ih
    area_d = (dxmax - dxmin) * (dymax - dymin)
    area_g = (gxmax - gxmin) * (gymax - gymin)
    union = (area_d + area_g) - inter
    iou = inter / union
    match = det_l_ref[...] == gt_l_ref[...]
    masked = jnp.where(match, iou, -1.0)
    part = jnp.max(masked, axis=0, keepdims=True)
    m_acc[...] = jnp.maximum(m_acc[...], part)

    @pl.when(i == nblk - 1)
    def _fin():
        m = m_acc[...]
        obj_idx = jax.lax.broadcasted_iota(jnp.int32, (1, _OBJ_PAD), 1)
        valid = (m >= 0.0) & (obj_idx < n_obj_ref[0])
        w = valid.astype(jnp.float32)
        n = jnp.sum(w, keepdims=True)
        s = jnp.sum(w * (1.0 - jnp.maximum(m, 0.0)), keepdims=True)
        out_ref[...] = (s / n).reshape(1, 1)


def kernel(det_boxes, det_scores, det_labels, boxes, labels):
    del det_scores
    db = det_boxes[0].astype(jnp.float32)
    dl = det_labels[0].astype(jnp.int32).reshape(-1, 1)
    gb = boxes[0].astype(jnp.float32)
    gl = labels[0].astype(jnp.int32)
    n_det = db.shape[0]
    n_obj = gb.shape[0]

    gt_c = jnp.zeros((4, _OBJ_PAD), jnp.float32).at[:, :n_obj].set(gb.T)
    gt_l = jnp.full((1, _OBJ_PAD), -1, jnp.int32).at[0, :n_obj].set(gl)
    n_obj_arr = jnp.full((1,), n_obj, jnp.int32)

    nblk = pl.cdiv(n_det, _DET_BLK)
    out = pl.pallas_call(
        _loss_kernel,
        grid=(nblk,),
        in_specs=[
            pl.BlockSpec(memory_space=pltpu.SMEM),
            pl.BlockSpec((_DET_BLK, 4), lambda i: (i, 0)),
            pl.BlockSpec((_DET_BLK, 1), lambda i: (i, 0)),
            pl.BlockSpec((4, _OBJ_PAD), lambda i: (0, 0)),
            pl.BlockSpec((1, _OBJ_PAD), lambda i: (0, 0)),
        ],
        out_specs=pl.BlockSpec((1, 1), lambda i: (0, 0)),
        out_shape=jax.ShapeDtypeStruct((1, 1), jnp.float32),
        scratch_shapes=[pltpu.VMEM((1, _OBJ_PAD), jnp.float32)],
    )(n_obj_arr, db, dl, gt_c, gt_l)
    return out[0, 0]


# speedup vs baseline: 3.1750x; 1.0064x over previous
import jax
import jax.numpy as jnp
from jax.experimental import pallas as pl
from jax.experimental.pallas import tpu as pltpu

_OBJ_PAD = 1024
_DET_BLK = 5000


def _loss_kernel(n_obj_ref, det_c_ref, det_l_ref, gt_c_ref, gt_l_ref,
                 out_ref, m_acc):
    i = pl.program_id(0)
    nblk = pl.num_programs(0)

    @pl.when(i == 0)
    def _init():
        m_acc[...] = jnp.full((1, _OBJ_PAD), -1.0, jnp.float32)

    dc = det_c_ref[...]
    dxmin = dc[:, 0:1]
    dymin = dc[:, 1:2]
    dxmax = dc[:, 2:3]
    dymax = dc[:, 3:4]
    gc = gt_c_ref[...]
    gxmin = gc[0:1, :]
    gymin = gc[1:2, :]
    gxmax = gc[2:3, :]
    gymax = gc[3:4, :]

    lox = jnp.maximum(dxmin, gxmin)
    loy = jnp.maximum(dymin, gymin)
    hix = jnp.minimum(dxmax, gxmax)
    hiy = jnp.minimum(dymax, gymax)
    iw = jnp.maximum(hix - lox, 0.0)
    ih = jnp.maximum(hiy - loy, 0.0)
    inter = iw * ih
    area_d = (dxmax - dxmin) * (dymax - dymin)
    area_g = (gxmax - gxmin) * (gymax - gymin)
    union = (area_d + area_g) - inter
    iou = inter / union
    match = det_l_ref[...] == gt_l_ref[...]
    masked = jnp.where(match, iou, -1.0)
    part = jnp.max(masked, axis=0, keepdims=True)
    m_acc[...] = jnp.maximum(m_acc[...], part)

    @pl.when(i == nblk - 1)
    def _fin():
        m = m_acc[...]
        obj_idx = jax.lax.broadcasted_iota(jnp.int32, (1, _OBJ_PAD), 1)
        valid = (m >= 0.0) & (obj_idx < n_obj_ref[0])
        w = valid.astype(jnp.float32)
        n = jnp.sum(w, keepdims=True)
        s = jnp.sum(w * (1.0 - jnp.maximum(m, 0.0)), keepdims=True)
        out_ref[...] = (s / n).reshape(1, 1)


def kernel(det_boxes, det_scores, det_labels, boxes, labels):
    del det_scores
    db = det_boxes[0].astype(jnp.float32)
    dl = det_labels[0].astype(jnp.int32).reshape(-1, 1)
    gb = boxes[0].astype(jnp.float32)
    gl = labels[0].astype(jnp.int32)
    n_det = db.shape[0]
    n_obj = gb.shape[0]

    gt_c = jnp.zeros((4, _OBJ_PAD), jnp.float32).at[:, :n_obj].set(gb.T)
    gt_l = jnp.full((1, _OBJ_PAD), -1, jnp.int32).at[0, :n_obj].set(gl)
    n_obj_arr = jnp.full((1,), n_obj, jnp.int32)

    nblk = pl.cdiv(n_det, _DET_BLK)
    out = pl.pallas_call(
        _loss_kernel,
        grid=(nblk,),
        in_specs=[
            pl.BlockSpec(memory_space=pltpu.SMEM),
            pl.BlockSpec((_DET_BLK, 4), lambda i: (i, 0)),
            pl.BlockSpec((_DET_BLK, 1), lambda i: (i, 0)),
            pl.BlockSpec((4, _OBJ_PAD), lambda i: (0, 0)),
            pl.BlockSpec((1, _OBJ_PAD), lambda i: (0, 0)),
        ],
        out_specs=pl.BlockSpec((1, 1), lambda i: (0, 0)),
        out_shape=jax.ShapeDtypeStruct((1, 1), jnp.float32),
        scratch_shapes=[pltpu.VMEM((1, _OBJ_PAD), jnp.float32)],
    )(n_obj_arr, db, dl, gt_c, gt_l)
    return out[0, 0]
